# Initial kernel scaffold; baseline (speedup 1.0000x reference)
import jax
import jax.numpy as jnp
from jax.experimental import pallas as pl


def kernel(reps, rep_table, centers, timestep):
    raise NotImplementedError("write your pallas kernel here")


# speedup vs baseline: 3.5283x; 3.5283x over previous
import functools

import jax
import jax.numpy as jnp
from jax.experimental import pallas as pl

_LAMB = 0.1


def _tile_kernel(rt_ref, centers_ref, val_ref, idx_ref, *, T, K, D):
    b = pl.program_id(0)
    rows = jax.lax.broadcasted_iota(jnp.int32, (T, 1), 0)
    jj = jax.lax.broadcasted_iota(jnp.int32, (T, T), 1)
    keep = (jj >= (T - 1 - rows)).astype(jnp.float32)
    rt = rt_ref[0]
    x = jnp.sum(rt * keep[:, :, None], axis=1)
    seg = rows.astype(jnp.float32) + 1.0
    x = x / seg
    c = centers_ref[...]
    scale = 1.0 / jnp.sqrt(jnp.float32(D))
    logits = jax.lax.dot_general(
        x, c, (((1,), (1,)), ((), ())), preferred_element_type=jnp.float32
    ) * scale
    m = jnp.max(logits, axis=1, keepdims=True)
    e = jnp.exp(logits - m)
    attn = e / jnp.sum(e, axis=1, keepdims=True)
    xq = jax.lax.dot_general(
        attn, c, (((1,), (0,)), ((), ())), preferred_element_type=jnp.float32
    )
    xx = jnp.sum(xq * xq, axis=1, keepdims=True)
    cc = jnp.sum(c * c, axis=1)
    xc = jax.lax.dot_general(
        xq, c, (((1,), (1,)), ((), ())), preferred_element_type=jnp.float32
    )
    loss = xx - 2.0 * xc + cc[None, :] + _LAMB * (1.0 - seg)
    val_ref[b, :] = jnp.min(loss, axis=1)
    idx_ref[b, :] = jnp.argmin(loss, axis=1).astype(jnp.int32)


def kernel(reps, rep_table, centers, timestep):
    B, T, D = reps.shape
    K = centers.shape[0]
    t = T
    start = timestep - t
    rt = jax.lax.dynamic_slice_in_dim(rep_table[:, :t], start, t, axis=2)
    val, idx = pl.pallas_call(
        functools.partial(_tile_kernel, T=T, K=K, D=D),
        grid=(B,),
        in_specs=[
            pl.BlockSpec((1, T, T, D), lambda b: (b, 0, 0, 0)),
            pl.BlockSpec((K, D), lambda b: (0, 0)),
        ],
        out_specs=[
            pl.BlockSpec((B, T), lambda b: (0, 0)),
            pl.BlockSpec((B, T), lambda b: (0, 0)),
        ],
        out_shape=[
            jax.ShapeDtypeStruct((B, T), jnp.float32),
            jax.ShapeDtypeStruct((B, T), jnp.int32),
        ],
    )(rt, centers)
    costs = jnp.full((B, T + 1), jnp.inf, jnp.float32)
    tokens = jnp.zeros((B, T + 1), jnp.int32)
    costs = jax.lax.dynamic_update_slice(costs, jnp.flip(val, axis=1), (0, start))
    tokens = jax.lax.dynamic_update_slice(tokens, jnp.flip(idx, axis=1), (0, start))
    return costs, tokens


# speedup vs baseline: 5.3705x; 1.5221x over previous
import functools

import jax
import jax.numpy as jnp
from jax.experimental import pallas as pl
from jax.experimental.pallas import tpu as pltpu

_LAMB = 0.1
_RC = 16


def _chunk_copy(rt_hbm, bufs, sems, bb, c):
    T = rt_hbm.shape[1]
    j0 = T - _RC * (c + 1)
    return pltpu.make_async_copy(
        rt_hbm.at[bb, pl.ds(_RC * c, _RC), pl.ds(j0, _RC * (c + 1)), :],
        bufs[c],
        sems.at[c],
    )


def _kernel(rt_hbm, centers_ref, val_ref, idx_ref, *bufs_sems, T, K, D, B, NC):
    bufs = bufs_sems[:NC]
    x_ref = bufs_sems[NC]
    sems = bufs_sems[NC + 1]
    b = pl.program_id(0)

    @pl.when(b == 0)
    def _prologue():
        for c in range(NC):
            _chunk_copy(rt_hbm, bufs, sems, 0, c).start()

    rr = jax.lax.broadcasted_iota(jnp.int32, (_RC, _RC), 0)
    jj = jax.lax.broadcasted_iota(jnp.int32, (_RC, _RC), 1)
    keep = (jj >= _RC - 1 - rr).astype(jnp.float32)[:, :, None]

    for c in range(NC):
        _chunk_copy(rt_hbm, bufs, sems, b, c).wait()
        buf = bufs[c][...]
        x_rows = jnp.sum(buf[:, :_RC, :] * keep, axis=1)
        if c > 0:
            x_rows = x_rows + jnp.sum(buf[:, _RC:, :], axis=1)
        x_ref[pl.ds(_RC * c, _RC), :] = x_rows

        @pl.when(b + 1 < B)
        def _next():
            _chunk_copy(rt_hbm, bufs, sems, b + 1, c).start()

    rows = jax.lax.broadcasted_iota(jnp.int32, (T, 1), 0)
    seg = rows.astype(jnp.float32) + 1.0
    x = x_ref[...] / seg
    c_ = centers_ref[...]
    scale = 1.0 / jnp.sqrt(jnp.float32(D))
    logits = jax.lax.dot_general(
        x, c_, (((1,), (1,)), ((), ())), preferred_element_type=jnp.float32
    ) * scale
    m = jnp.max(logits, axis=1, keepdims=True)
    e = jnp.exp(logits - m)
    attn = e / jnp.sum(e, axis=1, keepdims=True)
    xq = jax.lax.dot_general(
        attn, c_, (((1,), (0,)), ((), ())), preferred_element_type=jnp.float32
    )
    xx = jnp.sum(xq * xq, axis=1, keepdims=True)
    cc = jnp.sum(c_ * c_, axis=1)
    xc = jax.lax.dot_general(
        xq, c_, (((1,), (1,)), ((), ())), preferred_element_type=jnp.float32
    )
    loss = xx - 2.0 * xc + cc[None, :] + _LAMB * (1.0 - seg)
    val_ref[b, :] = jnp.min(loss, axis=1)
    idx_ref[b, :] = jnp.argmin(loss, axis=1).astype(jnp.int32)


def kernel(reps, rep_table, centers, timestep):
    B, T, D = reps.shape
    K = centers.shape[0]
    t = T
    start = timestep - t
    rt = jax.lax.dynamic_slice_in_dim(rep_table[:, :t], start, t, axis=2)
    NC = T // _RC
    val, idx = pl.pallas_call(
        functools.partial(_kernel, T=T, K=K, D=D, B=B, NC=NC),
        grid=(B,),
        in_specs=[
            pl.BlockSpec(memory_space=pl.ANY),
            pl.BlockSpec((K, D), lambda b: (0, 0)),
        ],
        out_specs=[
            pl.BlockSpec((B, T), lambda b: (0, 0)),
            pl.BlockSpec((B, T), lambda b: (0, 0)),
        ],
        out_shape=[
            jax.ShapeDtypeStruct((B, T), jnp.float32),
            jax.ShapeDtypeStruct((B, T), jnp.int32),
        ],
        scratch_shapes=(
            [pltpu.VMEM((_RC, _RC * (c + 1), D), jnp.float32) for c in range(NC)]
            + [pltpu.VMEM((T, D), jnp.float32), pltpu.SemaphoreType.DMA((NC,))]
        ),
    )(rt, centers)
    costs = jnp.full((B, T + 1), jnp.inf, jnp.float32)
    tokens = jnp.zeros((B, T + 1), jnp.int32)
    costs = jax.lax.dynamic_update_slice(costs, jnp.flip(val, axis=1), (0, start))
    tokens = jax.lax.dynamic_update_slice(tokens, jnp.flip(idx, axis=1), (0, start))
    return costs, tokens
